# Initial kernel scaffold; baseline (speedup 1.0000x reference)
import jax
import jax.numpy as jnp
from jax.experimental import pallas as pl


def kernel(h, edge_index, Wp1, bp1, Ws1, Wn1, b1, Wp2, bp2, Ws2, Wn2, b2):
    raise NotImplementedError("write your pallas kernel here")


# speedup vs baseline: 1.0593x; 1.0593x over previous
import functools

import jax
import jax.numpy as jnp
from jax import lax
from jax.experimental import pallas as pl
from jax.experimental.pallas import tpu as pltpu

N = 10000
E = 320000
IN_DIM = 128
HID = 128
CLS = 32

_PREC = lax.Precision.HIGHEST


def _pre_body(h_ref, wp_ref, bp_ref, ws_ref, m_ref, hs_ref):
    h = h_ref[...]
    m_ref[...] = jnp.maximum(
        jnp.dot(h, wp_ref[...].T, precision=_PREC) + bp_ref[...], 0.0)
    hs_ref[...] = jnp.dot(h, ws_ref[...].T, precision=_PREC)


def _mid_body(hs_ref, p_ref, wn_ref, b_ref, wp2_ref, bp2_ref, ws2_ref,
              m2_ref, hs2_ref):
    x = hs_ref[...] + jnp.dot(p_ref[...], wn_ref[...].T, precision=_PREC) + b_ref[...]
    h1 = jnp.where(x > 0, x, jnp.exp(jnp.minimum(x, 0.0)) - 1.0)
    m2_ref[...] = jnp.maximum(
        jnp.dot(h1, wp2_ref[...].T, precision=_PREC) + bp2_ref[...], 0.0)
    hs2_ref[...] = jnp.dot(h1, ws2_ref[...].T, precision=_PREC)


def _post_body(hs2_ref, p2_ref, wn2_ref, b2_ref, out_ref):
    logits = (hs2_ref[...] + jnp.dot(p2_ref[...], wn2_ref[...].T, precision=_PREC)
              + b2_ref[...])
    out_ref[...] = jnp.mean(logits, axis=1, keepdims=True)


def _segment_max(m, src, dst):
    pooled = jax.ops.segment_max(m[src], dst, num_segments=N)
    return jnp.where(jnp.isfinite(pooled), pooled, 0.0)


def kernel(h, edge_index, Wp1, bp1, Ws1, Wn1, b1, Wp2, bp2, Ws2, Wn2, b2):
    src = edge_index[0].astype(jnp.int32)
    dst = edge_index[1].astype(jnp.int32)

    m1, hs1 = pl.pallas_call(
        _pre_body,
        out_shape=[jax.ShapeDtypeStruct((N, IN_DIM), jnp.float32),
                   jax.ShapeDtypeStruct((N, HID), jnp.float32)],
    )(h, Wp1, bp1, Ws1)

    pooled1 = _segment_max(m1, src, dst)

    m2, hs2 = pl.pallas_call(
        _mid_body,
        out_shape=[jax.ShapeDtypeStruct((N, HID), jnp.float32),
                   jax.ShapeDtypeStruct((N, CLS), jnp.float32)],
    )(hs1, pooled1, Wn1, b1, Wp2, bp2, Ws2)

    pooled2 = _segment_max(m2, src, dst)

    out = pl.pallas_call(
        _post_body,
        out_shape=jax.ShapeDtypeStruct((N, 1), jnp.float32),
    )(hs2, pooled2, Wn2, b2)
    return out.reshape(N)
